# baseline (speedup 1.0000x reference)
import functools

import jax
import jax.numpy as jnp
from jax import lax
from jax.experimental import pallas as pl
from jax.experimental.pallas import tpu as pltpu
from jax.experimental.pallas import tpu_sc as plsc

_THRESH = 0.02


def _conf_body(x_ref, eh_ref, ridx_ref, st_ref):
    i = pl.program_id(0)
    h = pl.program_id(1)
    nh = pl.num_programs(1)
    BB = x_ref.shape[1]
    Bn = BB * pl.num_programs(0)
    x = x_ref[0]
    m = jnp.max(x, axis=-1, keepdims=True)
    s = jnp.sum(jnp.exp(x - m), axis=-1)
    conf = 1.0 / s
    ex = conf >= jnp.float32(_THRESH)
    last = jnp.int32(nh)

    @pl.when(h == 0)
    def _():
        st_ref[...] = jnp.where(ex, jnp.int32(0), last)

    @pl.when(h > 0)
    def _():
        cur = st_ref[...]
        st_ref[...] = jnp.where((cur == last) & ex, h.astype(jnp.int32), cur)

    @pl.when(h == nh - 1)
    def _():
        eh = st_ref[...]
        bloc = jax.lax.broadcasted_iota(jnp.int32, (1, BB), 1)[0]
        eh_ref[...] = eh
        ridx_ref[...] = eh * Bn + i * BB + bloc


def _exit_heads(logits):
    Hn, Bn, Cn = logits.shape
    BB = 2048
    return pl.pallas_call(
        _conf_body,
        grid=(Bn // BB, Hn - 1),
        in_specs=[pl.BlockSpec((1, BB, Cn), lambda i, h: (h, i, 0))],
        out_specs=[
            pl.BlockSpec((BB,), lambda i, h: (i,)),
            pl.BlockSpec((BB,), lambda i, h: (i,)),
        ],
        out_shape=[
            jax.ShapeDtypeStruct((Bn,), jnp.int32),
            jax.ShapeDtypeStruct((Bn,), jnp.int32),
        ],
        scratch_shapes=[pltpu.VMEM((BB,), jnp.int32)],
    )(logits)


def _make_sc_gather(Bn, Cn, dtype):
    info = plsc.get_sparse_core_info()
    NW = info.num_cores * info.num_subcores
    rows_per_w = Bn // NW
    CH = 32
    n_ch = rows_per_w // CH
    mesh = plsc.VectorSubcoreMesh(core_axis_name="c", subcore_axis_name="s")

    @functools.partial(
        pl.kernel,
        mesh=mesh,
        out_type=jax.ShapeDtypeStruct((Bn, Cn), dtype),
        scratch_types=[
            pltpu.VMEM((rows_per_w,), jnp.int32),
            pltpu.VMEM((2, CH, Cn), dtype),
            pltpu.SemaphoreType.DMA,
            pltpu.SemaphoreType.DMA,
        ],
    )
    def sc_gather(table_hbm, ridx_hbm, out_hbm, idx_v, rows_v, gsem, osem):
        wid = lax.axis_index("s") * info.num_cores + lax.axis_index("c")
        base = wid * rows_per_w
        pltpu.sync_copy(ridx_hbm.at[pl.ds(base, rows_per_w)], idx_v)

        def fill(j, slot):
            gets = []
            for g in range(CH // 16):
                vec = idx_v[pl.ds(j * CH + g * 16, 16)]
                for r in range(16):
                    row = vec[r]
                    gets.append(
                        pltpu.async_copy(
                            table_hbm.at[pl.ds(row, 1)],
                            rows_v.at[slot, pl.ds(g * 16 + r, 1)],
                            gsem,
                        )
                    )
            for h in gets:
                h.wait()

        def body(i, carry):
            j0 = i * 2
            fill(j0, 0)
            h0 = pltpu.async_copy(
                rows_v.at[0], out_hbm.at[pl.ds(base + j0 * CH, CH)], osem
            )
            fill(j0 + 1, 1)
            h0.wait()
            pltpu.async_copy(
                rows_v.at[1], out_hbm.at[pl.ds(base + (j0 + 1) * CH, CH)], osem
            ).wait()
            return carry

        lax.fori_loop(0, n_ch // 2, body, 0)

    return sc_gather


def kernel(logits):
    Hn, Bn, Cn = logits.shape
    eh, ridx = _exit_heads(logits)
    table = logits.reshape(Hn * Bn, Cn)
    out = _make_sc_gather(Bn, Cn, logits.dtype)(table, ridx)
    return out, eh


# speedup vs baseline: 1.0236x; 1.0236x over previous
import functools

import jax
import jax.numpy as jnp
from jax import lax
from jax.experimental import pallas as pl
from jax.experimental.pallas import tpu as pltpu
from jax.experimental.pallas import tpu_sc as plsc

_THRESH = 0.02


def _conf_body(x0_ref, x1_ref, x2_ref, eh_ref, ridx_ref):
    i = pl.program_id(0)
    BB = x0_ref.shape[1]
    Bn = BB * pl.num_programs(0)
    Hm1 = 3
    exs = []
    for r in (x0_ref, x1_ref, x2_ref):
        x = r[0]
        m = jnp.max(x, axis=-1, keepdims=True)
        s = jnp.sum(jnp.exp(x - m), axis=-1)
        conf = 1.0 / s
        exs.append(conf >= jnp.float32(_THRESH))
    eh = jnp.full((BB,), Hm1, jnp.int32)
    for h in range(Hm1 - 1, -1, -1):
        eh = jnp.where(exs[h], jnp.int32(h), eh)
    bloc = jax.lax.broadcasted_iota(jnp.int32, (1, BB), 1)[0]
    eh_ref[...] = eh
    ridx_ref[...] = eh * Bn + i * BB + bloc


def _exit_heads(logits):
    Hn, Bn, Cn = logits.shape
    BB = 1024
    specs = [
        pl.BlockSpec((1, BB, Cn), functools.partial(lambda h, i: (h, i, 0), h))
        for h in range(Hn - 1)
    ]
    return pl.pallas_call(
        _conf_body,
        grid=(Bn // BB,),
        in_specs=specs,
        out_specs=[
            pl.BlockSpec((BB,), lambda i: (i,)),
            pl.BlockSpec((BB,), lambda i: (i,)),
        ],
        out_shape=[
            jax.ShapeDtypeStruct((Bn,), jnp.int32),
            jax.ShapeDtypeStruct((Bn,), jnp.int32),
        ],
    )(logits, logits, logits)


def _make_sc_gather(Bn, Cn, dtype):
    info = plsc.get_sparse_core_info()
    NW = info.num_cores * info.num_subcores
    rows_per_w = Bn // NW
    CH = 32
    n_ch = rows_per_w // CH
    mesh = plsc.VectorSubcoreMesh(core_axis_name="c", subcore_axis_name="s")

    @functools.partial(
        pl.kernel,
        mesh=mesh,
        out_type=jax.ShapeDtypeStruct((Bn, Cn), dtype),
        scratch_types=[
            pltpu.VMEM((rows_per_w,), jnp.int32),
            pltpu.VMEM((2, CH, Cn), dtype),
            pltpu.SemaphoreType.DMA,
            pltpu.SemaphoreType.DMA,
        ],
    )
    def sc_gather(table_hbm, ridx_hbm, out_hbm, idx_v, rows_v, gsem, osem):
        wid = lax.axis_index("s") * info.num_cores + lax.axis_index("c")
        base = wid * rows_per_w
        pltpu.sync_copy(ridx_hbm.at[pl.ds(base, rows_per_w)], idx_v)

        def fill(j, slot):
            gets = []
            for g in range(CH // 16):
                vec = idx_v[pl.ds(j * CH + g * 16, 16)]
                for r in range(16):
                    row = vec[r]
                    gets.append(
                        pltpu.async_copy(
                            table_hbm.at[pl.ds(row, 1)],
                            rows_v.at[slot, pl.ds(g * 16 + r, 1)],
                            gsem,
                        )
                    )
            for h in gets:
                h.wait()

        def body(i, carry):
            j0 = i * 2
            fill(j0, 0)
            h0 = pltpu.async_copy(
                rows_v.at[0], out_hbm.at[pl.ds(base + j0 * CH, CH)], osem
            )
            fill(j0 + 1, 1)
            h0.wait()
            pltpu.async_copy(
                rows_v.at[1], out_hbm.at[pl.ds(base + (j0 + 1) * CH, CH)], osem
            ).wait()
            return carry

        lax.fori_loop(0, n_ch // 2, body, 0)

    return sc_gather


def kernel(logits):
    Hn, Bn, Cn = logits.shape
    eh, ridx = _exit_heads(logits)
    table = logits.reshape(Hn * Bn, Cn)
    out = _make_sc_gather(Bn, Cn, logits.dtype)(table, ridx)
    return out, eh
